# Initial kernel scaffold; baseline (speedup 1.0000x reference)
import jax
import jax.numpy as jnp
from jax.experimental import pallas as pl


def kernel(reconstructed_3d, input_2d, slice_idx):
    raise NotImplementedError("write your pallas kernel here")


# speedup vs baseline: 4.5194x; 4.5194x over previous
import jax
import jax.numpy as jnp
from jax import lax
from jax.experimental import pallas as pl
from jax.experimental.pallas import tpu as pltpu
from jax.experimental.pallas import tpu_sc as plsc

NC, NS, L = 2, 16, 16
NW = NC * NS
B, H, W, D = 8, 256, 256, 128
PER_SAMPLE = H * W
TOTAL = B * PER_SAMPLE
CHUNK = TOTAL // NW
ROWS = CHUNK // L


def _body(r3d_hbm, in2d_hbm, idx_hbm, out_hbm,
          idx16_v, idxbuf_v, gbuf_v, ybuf_v, acc_v, gsem, ysem):
    c = lax.axis_index("c")
    s = lax.axis_index("s")
    wid = s * NC + c

    ycopy = pltpu.async_copy(in2d_hbm.at[pl.ds(wid * CHUNK, CHUNK)], ybuf_v, ysem)

    pltpu.sync_copy(idx_hbm.at[wid], idx16_v)
    lanes = lax.broadcasted_iota(jnp.int32, (L,), 0)
    base = idx16_v[...] + lanes * D + wid * (CHUNK * D)

    def build(t, carry):
        idxbuf_v[pl.ds(t * L, L)] = base + t * (L * D)
        return carry
    lax.fori_loop(0, ROWS, build, 0, unroll=8)

    gcopy = pltpu.async_copy(r3d_hbm.at[idxbuf_v], gbuf_v, gsem)
    gcopy.wait()
    ycopy.wait()

    def red(t, acc):
        d = gbuf_v[pl.ds(t * L, L)] - ybuf_v[pl.ds(t * L, L)]
        return acc + d * d
    acc = lax.fori_loop(0, ROWS, red, jnp.zeros((L,), jnp.float32), unroll=8)
    acc_v[...] = acc
    pltpu.sync_copy(acc_v, out_hbm.at[wid])


def kernel(reconstructed_3d, input_2d, slice_idx):
    r3d_flat = reconstructed_3d.reshape(-1)
    in2d = input_2d.reshape(-1)
    idx = jnp.broadcast_to(
        slice_idx.astype(jnp.int32)[:, None, None], (B, NW // B, L)
    ).reshape(NW, L)
    mesh = plsc.VectorSubcoreMesh(core_axis_name="c", subcore_axis_name="s")
    partials = pl.kernel(
        _body,
        out_type=jax.ShapeDtypeStruct((NW, L), jnp.float32),
        mesh=mesh,
        scratch_types=[
            pltpu.VMEM((L,), jnp.int32),
            pltpu.VMEM((CHUNK,), jnp.int32),
            pltpu.VMEM((CHUNK,), jnp.float32),
            pltpu.VMEM((CHUNK,), jnp.float32),
            pltpu.VMEM((L,), jnp.float32),
            pltpu.SemaphoreType.DMA,
            pltpu.SemaphoreType.DMA,
        ],
    )(r3d_flat, in2d, idx)
    return jnp.sum(partials) / TOTAL
